# Initial kernel scaffold; baseline (speedup 1.0000x reference)
import jax
import jax.numpy as jnp
from jax.experimental import pallas as pl


def kernel(x, W):
    raise NotImplementedError("write your pallas kernel here")


# speedup vs baseline: 3.4665x; 3.4665x over previous
import functools
import math

import jax
import jax.numpy as jnp
import numpy as np
from jax import lax
from jax.experimental import pallas as pl
from jax.experimental.pallas import tpu as pltpu
from jax.experimental.pallas import tpu_sc as plsc

_D_MODEL = 100000
_DIM = 64
_BATCH = 4096
_SEQ = 200
_B = _BATCH * _SEQ

_NC = 2
_NS = 16
_NW = _NC * _NS
_B_PER_W = _B // _NW
_GROUP = 128
_NGROUP = _B_PER_W // _GROUP

_SCALE = np.float32(math.sqrt(_D_MODEL))


def _scale_block(w_ref, o_ref):
    o_ref[...] = w_ref[...] * _SCALE


def _scale_table(w2d):
    rows = w2d.shape[0]
    blk = rows // 10
    return pl.pallas_call(
        _scale_block,
        out_shape=jax.ShapeDtypeStruct(w2d.shape, jnp.float32),
        grid=(10,),
        in_specs=[pl.BlockSpec((blk, 128), lambda i: (i, 0))],
        out_specs=pl.BlockSpec((blk, 128), lambda i: (i, 0)),
    )(w2d)


_sc_mesh = plsc.VectorSubcoreMesh(core_axis_name="c", subcore_axis_name="s")


@functools.partial(
    pl.kernel,
    mesh=_sc_mesh,
    out_type=jax.ShapeDtypeStruct((_B, _DIM), jnp.float32),
    scratch_types=[
        pltpu.VMEM((_NGROUP, _GROUP), jnp.int32),
        pltpu.VMEM((_GROUP, _DIM), jnp.float32),
        pltpu.SemaphoreType.DMA,
    ],
    compiler_params=pltpu.CompilerParams(use_tc_tiling_on_sc=False),
)
def _sc_gather(w_hbm, idx_hbm, out_hbm, idx_v, rows_v, gsem):
    wid = lax.axis_index("s") * _NC + lax.axis_index("c")
    pltpu.sync_copy(idx_hbm.at[wid], idx_v)
    base = wid * _B_PER_W

    def step(j, carry):
        pltpu.async_copy(w_hbm.at[idx_v.at[j]], rows_v, gsem).wait()
        pltpu.sync_copy(rows_v, out_hbm.at[pl.ds(base + j * _GROUP, _GROUP)])
        return carry

    lax.fori_loop(0, _NGROUP, step, 0)


def kernel(x, W):
    ws = _scale_table(W.reshape(_D_MODEL // 2, 128)).reshape(_D_MODEL, _DIM)
    idx = x.reshape(_NW, _NGROUP, _GROUP)
    out = _sc_gather(ws, idx)
    return out.reshape(_BATCH, _SEQ, _DIM)


# speedup vs baseline: 4.1508x; 1.1974x over previous
import functools
import math

import jax
import jax.numpy as jnp
import numpy as np
from jax import lax
from jax.experimental import pallas as pl
from jax.experimental.pallas import tpu as pltpu
from jax.experimental.pallas import tpu_sc as plsc

_D_MODEL = 100000
_DIM = 64
_BATCH = 4096
_SEQ = 200
_B = _BATCH * _SEQ

_NC = 2
_NS = 16
_NW = _NC * _NS
_B_PER_W = _B // _NW
_GROUP = 128
_NGROUP = _B_PER_W // _GROUP

_SCALE = np.float32(math.sqrt(_D_MODEL))


def _scale_block(w_ref, o_ref):
    o_ref[...] = w_ref[...] * _SCALE


def _scale_table(w2d):
    rows = w2d.shape[0]
    blk = rows // 10
    return pl.pallas_call(
        _scale_block,
        out_shape=jax.ShapeDtypeStruct(w2d.shape, jnp.float32),
        grid=(10,),
        in_specs=[pl.BlockSpec((blk, 128), lambda i: (i, 0))],
        out_specs=pl.BlockSpec((blk, 128), lambda i: (i, 0)),
    )(w2d)


_sc_mesh = plsc.VectorSubcoreMesh(core_axis_name="c", subcore_axis_name="s")


_NBUF = 8
_LAG = 4


@functools.partial(
    pl.kernel,
    mesh=_sc_mesh,
    out_type=jax.ShapeDtypeStruct((_B, _DIM), jnp.float32),
    scratch_types=[
        pltpu.VMEM((_NGROUP, _GROUP), jnp.int32),
        pltpu.VMEM((_NBUF, _GROUP, _DIM), jnp.float32),
    ] + [pltpu.SemaphoreType.DMA] * (2 * _NBUF),
    compiler_params=pltpu.CompilerParams(use_tc_tiling_on_sc=False),
)
def _sc_gather(w_hbm, idx_hbm, out_hbm, idx_v, rows_v, *sems):
    gsem = sems[:_NBUF]
    osem = sems[_NBUF:]
    wid = lax.axis_index("s") * _NC + lax.axis_index("c")
    pltpu.sync_copy(idx_hbm.at[wid], idx_v)
    base = wid * _B_PER_W

    def out_slice(j):
        return out_hbm.at[pl.ds(base + j * _GROUP, _GROUP)]

    def issue_gather(j, b):
        pltpu.async_copy(w_hbm.at[idx_v.at[j]], rows_v.at[b], gsem[b])

    def wait_gather(b):
        pltpu.make_async_copy(
            w_hbm.at[pl.ds(0, _GROUP)], rows_v.at[b], gsem[b]).wait()

    def issue_out(j, b):
        pltpu.async_copy(rows_v.at[b], out_slice(j), osem[b])

    def wait_out(j, b):
        pltpu.make_async_copy(rows_v.at[b], out_slice(j), osem[b]).wait()

    def body(g, carry):
        for b in range(_NBUF):
            j = g * _NBUF + b

            @pl.when(g >= 1)
            def _():
                wait_out(j - _NBUF, b)

            issue_gather(j, b)

            bi = (b - _LAG) % _NBUF
            if b >= _LAG:
                wait_gather(bi)
                issue_out(j - _LAG, bi)
            else:
                @pl.when(g >= 1)
                def _():
                    wait_gather(bi)
                    issue_out(j - _LAG, bi)
        return carry

    lax.fori_loop(0, _NGROUP // _NBUF, body, 0)

    for j in range(_NGROUP - _LAG, _NGROUP):
        b = j % _NBUF
        wait_gather(b)
        issue_out(j, b)
    for j in range(_NGROUP - _NBUF, _NGROUP):
        wait_out(j, j % _NBUF)


def kernel(x, W):
    ws = _scale_table(W.reshape(_D_MODEL // 2, 128)).reshape(_D_MODEL, _DIM)
    idx = x.reshape(_NW, _NGROUP, _GROUP)
    out = _sc_gather(ws, idx)
    return out.reshape(_BATCH, _SEQ, _DIM)
